# Initial kernel scaffold; baseline (speedup 1.0000x reference)
import jax
import jax.numpy as jnp
from jax.experimental import pallas as pl


def kernel(x, edge_index, W1, b1, W2, b2):
    raise NotImplementedError("write your pallas kernel here")


# speedup vs baseline: 1.3275x; 1.3275x over previous
import functools

import jax
import jax.numpy as jnp
from jax import lax
from jax.experimental import pallas as pl
from jax.experimental.pallas import tpu as pltpu
from jax.experimental.pallas import tpu_sc as plsc

N = 10000
E = 320000
D = 128

NC, NS = 2, 16
NW = NC * NS
LANES = 16


_BN = 2000


def _proj_body(x_ref, w1_ref, b1_ref, p_ref, q_ref):
    w1a = w1_ref[:D, :]
    w1b = w1_ref[D:, :]
    x = x_ref[...]
    p_ref[...] = (
        jnp.dot(x, w1a - w1b, preferred_element_type=jnp.float32) + b1_ref[...]
    )
    q_ref[...] = jnp.dot(x, w1b, preferred_element_type=jnp.float32)


def _project(x, w1, b1):
    grid = (N // _BN,)
    return pl.pallas_call(
        _proj_body,
        grid=grid,
        in_specs=[
            pl.BlockSpec((_BN, D), lambda i: (i, 0)),
            pl.BlockSpec((2 * D, D), lambda i: (0, 0)),
            pl.BlockSpec((1, D), lambda i: (0, 0)),
        ],
        out_specs=[
            pl.BlockSpec((_BN, D), lambda i: (i, 0)),
            pl.BlockSpec((_BN, D), lambda i: (i, 0)),
        ],
        out_shape=[
            jax.ShapeDtypeStruct((N, D), jnp.float32),
            jax.ShapeDtypeStruct((N, D), jnp.float32),
        ],
    )(x, w1, b1.reshape(1, D))


_GCH = 200
_EPW = E // NW
_NGCH = _EPW // _GCH


def _gather_body(p_hbm, q_hbm, dst_hbm, src_hbm, pd_hbm, qs_hbm,
                 didx, sidx, pbuf, qbuf, sem_p, sem_q):
    wid = lax.axis_index("s") * NC + lax.axis_index("c")
    ebase = wid * _EPW

    def chunk(i, _):
        base = ebase + i * _GCH
        pltpu.sync_copy(dst_hbm.at[pl.ds(base, _GCH)], didx)
        pltpu.sync_copy(src_hbm.at[pl.ds(base, _GCH)], sidx)
        cp = pltpu.async_copy(p_hbm.at[didx], pbuf, sem_p)
        cq = pltpu.async_copy(q_hbm.at[sidx], qbuf, sem_q)
        cp.wait()
        cq.wait()
        pltpu.sync_copy(pbuf, pd_hbm.at[pl.ds(base, _GCH)])
        pltpu.sync_copy(qbuf, qs_hbm.at[pl.ds(base, _GCH)])
        return _

    lax.fori_loop(0, _NGCH, chunk, 0)


_gather = functools.partial(
    pl.kernel,
    mesh=plsc.VectorSubcoreMesh(
        core_axis_name="c", subcore_axis_name="s", num_cores=NC, num_subcores=NS
    ),
    out_type=[
        jax.ShapeDtypeStruct((E, D), jnp.float32),
        jax.ShapeDtypeStruct((E, D), jnp.float32),
    ],
    scratch_types=[
        pltpu.VMEM((_GCH,), jnp.int32),
        pltpu.VMEM((_GCH,), jnp.int32),
        pltpu.VMEM((_GCH, D), jnp.float32),
        pltpu.VMEM((_GCH, D), jnp.float32),
        pltpu.SemaphoreType.DMA,
        pltpu.SemaphoreType.DMA,
    ],
    compiler_params=pltpu.CompilerParams(needs_layout_passes=False),
)(_gather_body)


_BE = 3200


def _mlp_body(pd_ref, qs_ref, w2_ref, b2_ref, z_ref):
    h = jnp.maximum(pd_ref[...] + qs_ref[...], 0.0)
    z_ref[...] = (
        jnp.dot(h, w2_ref[...], preferred_element_type=jnp.float32) + b2_ref[...]
    )


def _edge_mlp(pd, qs, w2, b2):
    grid = (E // _BE,)
    return pl.pallas_call(
        _mlp_body,
        grid=grid,
        in_specs=[
            pl.BlockSpec((_BE, D), lambda i: (i, 0)),
            pl.BlockSpec((_BE, D), lambda i: (i, 0)),
            pl.BlockSpec((D, D), lambda i: (0, 0)),
            pl.BlockSpec((1, D), lambda i: (0, 0)),
        ],
        out_specs=pl.BlockSpec((_BE, D), lambda i: (i, 0)),
        out_shape=jax.ShapeDtypeStruct((E, D), jnp.float32),
    )(pd, qs, w2, b2.reshape(1, D))


_RPW = 320
_DCH = 2000
_NDCH = E // _DCH
_VPC = _DCH // LANES
_GB = 128
_FLUSH = _GB - LANES

_NEG = float("-inf")


def _scatter_body(z_hbm, dst_hbm, out_hbm, dbuf, rows_v, eids_v, zbuf, acc, sem):
    wid = lax.axis_index("s") * NC + lax.axis_index("c")
    nbase = wid * _RPW

    def init_row(r, _):
        for c in range(D // LANES):
            acc[r, pl.ds(c * LANES, LANES)] = jnp.full((LANES,), _NEG, jnp.float32)
        return _
    lax.fori_loop(0, _RPW, init_row, 0)
    for g in range(_GB // LANES):
        eids_v[pl.ds(g * LANES, LANES)] = jnp.zeros((LANES,), jnp.int32)
        rows_v[pl.ds(g * LANES, LANES)] = jnp.zeros((LANES,), jnp.int32)

    def flush(n):
        pltpu.async_copy(z_hbm.at[eids_v], zbuf, sem).wait()

        def upd(k, _):
            r = rows_v[pl.ds(k, LANES)][0]
            for c in range(D // LANES):
                sl = pl.ds(c * LANES, LANES)
                acc[r, sl] = jnp.maximum(acc[r, sl], zbuf[k, sl])
            return _
        lax.fori_loop(0, n, upd, 0)
        return jnp.int32(0)

    def chunk(i, nacc):
        pltpu.sync_copy(dst_hbm.at[pl.ds(i * _DCH, _DCH)], dbuf)

        def vec(j, nacc):
            d = dbuf[pl.ds(j * LANES, LANES)]
            r = d - nbase
            m = (r >= 0) & (r < _RPW)
            cnt = plsc.all_reduce_population_count(m)[0]

            def append(nacc):
                eid = lax.iota(jnp.int32, LANES) + (i * _DCH + j * LANES)
                plsc.store_compressed(rows_v.at[pl.ds(nacc, LANES)], r, mask=m)
                plsc.store_compressed(eids_v.at[pl.ds(nacc, LANES)], eid, mask=m)
                return nacc + cnt

            nacc = lax.cond(cnt > 0, append, lambda nacc: nacc, nacc)
            nacc = lax.cond(nacc >= _FLUSH, flush, lambda nacc: nacc, nacc)
            return nacc

        return lax.fori_loop(0, _VPC, vec, nacc)

    nacc = lax.fori_loop(0, _NDCH, chunk, jnp.int32(0))
    lax.cond(nacc > 0, flush, lambda nacc: jnp.int32(0), nacc)

    def fix_row(r, _):
        for c in range(D // LANES):
            sl = pl.ds(c * LANES, LANES)
            v = acc[r, sl]
            acc[r, sl] = jnp.where(v == _NEG, jnp.float32(0.0), v)
        return _
    lax.fori_loop(0, _RPW, fix_row, 0)

    @pl.when(wid < NW - 1)
    def _():
        pltpu.sync_copy(acc, out_hbm.at[pl.ds(nbase, _RPW)])

    @pl.when(wid == NW - 1)
    def _():
        rem = N - (NW - 1) * _RPW
        pltpu.sync_copy(acc.at[pl.ds(0, rem)], out_hbm.at[pl.ds(nbase, rem)])


_scatter = functools.partial(
    pl.kernel,
    mesh=plsc.VectorSubcoreMesh(
        core_axis_name="c", subcore_axis_name="s", num_cores=NC, num_subcores=NS
    ),
    out_type=jax.ShapeDtypeStruct((N, D), jnp.float32),
    scratch_types=[
        pltpu.VMEM((_DCH,), jnp.int32),
        pltpu.VMEM((_GB + LANES,), jnp.int32),
        pltpu.VMEM((_GB,), jnp.int32),
        pltpu.VMEM((_GB, D), jnp.float32),
        pltpu.VMEM((_RPW, D), jnp.float32),
        pltpu.SemaphoreType.DMA,
    ],
    compiler_params=pltpu.CompilerParams(needs_layout_passes=False),
)(_scatter_body)


@jax.jit
def kernel(x, edge_index, W1, b1, W2, b2):
    ei = edge_index.astype(jnp.int32)
    src = ei[0]
    dst = ei[1]
    p, q = _project(x, W1, b1)
    pd, qs = _gather(p, q, dst, src)
    z = _edge_mlp(pd, qs, W2, b2)
    return _scatter(z, dst)
